# Initial kernel scaffold; baseline (speedup 1.0000x reference)
import jax
import jax.numpy as jnp
from jax.experimental import pallas as pl


def kernel(x, w_router, w_gate, w_up, w_down):
    raise NotImplementedError("write your pallas kernel here")


# speedup vs baseline: 1.3400x; 1.3400x over previous
import functools

import jax
import jax.numpy as jnp
from jax.experimental import pallas as pl
from jax.experimental.pallas import tpu as pltpu

B, S, H, F, E, K = 2, 2048, 1024, 2688, 8, 2
T = B * S
A = T * K
AUX_W = 0.01

BR = 256
P_MAX = A + E * BR
NB = P_MAX // BR
NF = 3
FB = F // NF

_INTERP = False


def _router_body(x_ref, wr_ref, idx_ref, val_ref, loss_ref):
    x = x_ref[...]
    wr = wr_ref[...]
    logits = jnp.dot(x, wr, preferred_element_type=jnp.float32)
    m = jnp.max(logits, axis=-1, keepdims=True)
    ex = jnp.exp(logits - m)
    p = ex / jnp.sum(ex, axis=-1, keepdims=True)

    iota = jax.lax.broadcasted_iota(jnp.int32, p.shape, 1)
    m1 = jnp.max(p, axis=-1, keepdims=True)
    i1 = jnp.min(jnp.where(p >= m1, iota, E), axis=-1, keepdims=True)
    pm = jnp.where(iota == i1, -1.0, p)
    m2 = jnp.max(pm, axis=-1, keepdims=True)
    i2 = jnp.min(jnp.where(pm >= m2, iota, E), axis=-1, keepdims=True)

    den = m1 + m2 + 1e-9
    idx_ref[...] = jnp.concatenate([i1, i2], axis=-1)
    val_ref[...] = jnp.concatenate([m1 / den, m2 / den], axis=-1)

    sum_p = jnp.sum(p, axis=0)
    load = ((iota == i1) | (iota == i2)).astype(jnp.float32)
    sum_load = jnp.sum(load, axis=0)
    loss = (E * AUX_W / (T * T)) * jnp.sum(sum_p * sum_load)
    loss_ref[...] = jnp.broadcast_to(loss, (1, 1))


def _router(x_flat, w_router):
    return pl.pallas_call(
        _router_body,
        out_shape=(
            jax.ShapeDtypeStruct((T, K), jnp.int32),
            jax.ShapeDtypeStruct((T, K), jnp.float32),
            jax.ShapeDtypeStruct((1, 1), jnp.float32),
        ),
        interpret=_INTERP,
    )(x_flat, w_router)


def _gemm_body(be_ref, nu_ref, xs_ref, wg_ref, wu_ref, wd_ref, sw_ref,
               out_ref, acc_ref):
    b = pl.program_id(0)
    nf = pl.program_id(1)

    @pl.when(nf == 0)
    def _():
        acc_ref[...] = jnp.zeros_like(acc_ref)

    @pl.when(b < nu_ref[0])
    def _():
        xs = xs_ref[...]
        gate = jnp.dot(xs, wg_ref[0], preferred_element_type=jnp.float32)
        up = jnp.dot(xs, wu_ref[0], preferred_element_type=jnp.float32)
        g = gate * jax.lax.logistic(gate) * up
        acc_ref[...] += jnp.dot(g, wd_ref[0], preferred_element_type=jnp.float32)

    @pl.when(nf == NF - 1)
    def _():
        out_ref[...] = acc_ref[...] * sw_ref[...]


def _gemm(x_sorted, sorted_w, block_expert, nused, w_gate, w_up, w_down):
    grid_spec = pltpu.PrefetchScalarGridSpec(
        num_scalar_prefetch=2,
        grid=(NB, NF),
        in_specs=[
            pl.BlockSpec((BR, H), lambda b, nf, be, nu: (b, 0)),
            pl.BlockSpec((1, H, FB), lambda b, nf, be, nu: (be[b], 0, nf)),
            pl.BlockSpec((1, H, FB), lambda b, nf, be, nu: (be[b], 0, nf)),
            pl.BlockSpec((1, FB, H), lambda b, nf, be, nu: (be[b], nf, 0)),
            pl.BlockSpec((BR, 1), lambda b, nf, be, nu: (b, 0)),
        ],
        out_specs=pl.BlockSpec((BR, H), lambda b, nf, be, nu: (b, 0)),
        scratch_shapes=[pltpu.VMEM((BR, H), jnp.float32)],
    )
    return pl.pallas_call(
        _gemm_body,
        grid_spec=grid_spec,
        out_shape=jax.ShapeDtypeStruct((P_MAX, H), jnp.float32),
        interpret=_INTERP,
    )(block_expert, nused, x_sorted, w_gate, w_up, w_down,
      sorted_w.reshape(P_MAX, 1))


def kernel(x, w_router, w_gate, w_up, w_down):
    x_flat = x.reshape(T, H)
    idx, val, loss = _router(x_flat, w_router)

    e_flat = idx.reshape(A)
    v_flat = val.reshape(A)
    oh = (e_flat[:, None] == jnp.arange(E)[None, :]).astype(jnp.int32)
    counts = oh.sum(axis=0)
    psz = ((counts + BR - 1) // BR) * BR
    gb = jnp.cumsum(psz) - psz
    rank = jnp.take_along_axis(jnp.cumsum(oh, axis=0), e_flat[:, None], 1)[:, 0] - 1
    pos = gb[e_flat] + rank
    sorted_tok = jnp.zeros((P_MAX,), jnp.int32).at[pos].set(
        jnp.arange(A, dtype=jnp.int32) // K)
    sorted_w = jnp.zeros((P_MAX,), jnp.float32).at[pos].set(v_flat)
    x_sorted = x_flat[sorted_tok]
    ends = jnp.cumsum(psz)
    bstart = jnp.arange(NB) * BR
    block_expert = jnp.minimum(
        (bstart[:, None] >= ends[None, :]).sum(axis=1), E - 1).astype(jnp.int32)
    nused = (jnp.sum(psz) // BR).astype(jnp.int32).reshape(1)

    x_out = _gemm(x_sorted, sorted_w, block_expert, nused, w_gate, w_up, w_down)

    pos2 = pos.reshape(T, K)
    out_flat = x_out[pos2[:, 0]] + x_out[pos2[:, 1]]
    return out_flat.reshape(B, S, H), loss[0, 0]


# speedup vs baseline: 1.5770x; 1.1769x over previous
import functools

import jax
import jax.numpy as jnp
from jax.experimental import pallas as pl
from jax.experimental.pallas import tpu as pltpu

B, S, H, F, E, K = 2, 2048, 1024, 2688, 8, 2
T = B * S
A = T * K
AUX_W = 0.01

BR = 256
P_MAX = A + E * BR
NB = P_MAX // BR

_INTERP = False


def _router_body(x_ref, wr_ref, idx_ref, val_ref, loss_ref):
    x = x_ref[...]
    wr = wr_ref[...]
    logits = jnp.dot(x, wr, preferred_element_type=jnp.float32)
    m = jnp.max(logits, axis=-1, keepdims=True)
    ex = jnp.exp(logits - m)
    p = ex / jnp.sum(ex, axis=-1, keepdims=True)

    iota = jax.lax.broadcasted_iota(jnp.int32, p.shape, 1)
    m1 = jnp.max(p, axis=-1, keepdims=True)
    i1 = jnp.min(jnp.where(p >= m1, iota, E), axis=-1, keepdims=True)
    pm = jnp.where(iota == i1, -1.0, p)
    m2 = jnp.max(pm, axis=-1, keepdims=True)
    i2 = jnp.min(jnp.where(pm >= m2, iota, E), axis=-1, keepdims=True)

    den = m1 + m2 + 1e-9
    idx_ref[...] = jnp.concatenate([i1, i2], axis=-1)
    val_ref[...] = jnp.concatenate([m1 / den, m2 / den], axis=-1)

    sum_p = jnp.sum(p, axis=0)
    load = ((iota == i1) | (iota == i2)).astype(jnp.float32)
    sum_load = jnp.sum(load, axis=0)
    loss = (E * AUX_W / (T * T)) * jnp.sum(sum_p * sum_load)
    loss_ref[...] = jnp.broadcast_to(loss, (1, 1))


def _router(x_flat, w_router):
    return pl.pallas_call(
        _router_body,
        out_shape=(
            jax.ShapeDtypeStruct((T, K), jnp.int32),
            jax.ShapeDtypeStruct((T, K), jnp.float32),
            jax.ShapeDtypeStruct((1, 1), jnp.float32),
        ),
        interpret=_INTERP,
    )(x_flat, w_router)


def _gemm_body(be_ref, nu_ref, xs_ref, wg_ref, wu_ref, wd_ref, sw_ref,
               out_ref):
    b = pl.program_id(0)

    @pl.when(b < nu_ref[0])
    def _():
        xs = xs_ref[...]
        gate = jnp.dot(xs, wg_ref[0], preferred_element_type=jnp.float32)
        up = jnp.dot(xs, wu_ref[0], preferred_element_type=jnp.float32)
        g = (gate * jax.lax.logistic(gate) * up).astype(jnp.bfloat16)
        acc = jnp.dot(g, wd_ref[0], preferred_element_type=jnp.float32)
        out_ref[...] = acc * sw_ref[...]


def _gemm(x_sorted, sorted_w, block_expert, nused, w_gate, w_up, w_down):
    grid_spec = pltpu.PrefetchScalarGridSpec(
        num_scalar_prefetch=2,
        grid=(NB,),
        in_specs=[
            pl.BlockSpec((BR, H), lambda b, be, nu: (b, 0)),
            pl.BlockSpec((1, H, F), lambda b, be, nu: (be[b], 0, 0)),
            pl.BlockSpec((1, H, F), lambda b, be, nu: (be[b], 0, 0)),
            pl.BlockSpec((1, F, H), lambda b, be, nu: (be[b], 0, 0)),
            pl.BlockSpec((BR, 1), lambda b, be, nu: (b, 0)),
        ],
        out_specs=pl.BlockSpec((BR, H), lambda b, be, nu: (b, 0)),
    )
    return pl.pallas_call(
        _gemm_body,
        grid_spec=grid_spec,
        out_shape=jax.ShapeDtypeStruct((P_MAX, H), jnp.float32),
        interpret=_INTERP,
    )(block_expert, nused, x_sorted.astype(jnp.bfloat16),
      w_gate.astype(jnp.bfloat16), w_up.astype(jnp.bfloat16),
      w_down.astype(jnp.bfloat16), sorted_w.reshape(P_MAX, 1))


def kernel(x, w_router, w_gate, w_up, w_down):
    x_flat = x.reshape(T, H)
    idx, val, loss = _router(x_flat, w_router)

    e_flat = idx.reshape(A)
    v_flat = val.reshape(A)
    oh = (e_flat[:, None] == jnp.arange(E)[None, :]).astype(jnp.int32)
    counts = oh.sum(axis=0)
    psz = ((counts + BR - 1) // BR) * BR
    gb = jnp.cumsum(psz) - psz
    rank = jnp.take_along_axis(jnp.cumsum(oh, axis=0), e_flat[:, None], 1)[:, 0] - 1
    pos = gb[e_flat] + rank
    sorted_tok = jnp.zeros((P_MAX,), jnp.int32).at[pos].set(
        jnp.arange(A, dtype=jnp.int32) // K)
    sorted_w = jnp.zeros((P_MAX,), jnp.float32).at[pos].set(v_flat)
    x_sorted = x_flat[sorted_tok]
    ends = jnp.cumsum(psz)
    bstart = jnp.arange(NB) * BR
    block_expert = jnp.minimum(
        (bstart[:, None] >= ends[None, :]).sum(axis=1), E - 1).astype(jnp.int32)
    nused = (jnp.sum(psz) // BR).astype(jnp.int32).reshape(1)

    x_out = _gemm(x_sorted, sorted_w, block_expert, nused, w_gate, w_up, w_down)

    pos2 = pos.reshape(T, K)
    out_flat = x_out[pos2[:, 0]] + x_out[pos2[:, 1]]
    return out_flat.reshape(B, S, H), loss[0, 0]
